# baseline (speedup 1.0000x reference)
import functools

import jax
import jax.numpy as jnp
from jax import lax
from jax.experimental import pallas as pl
from jax.experimental.pallas import tpu as pltpu
from jax.experimental.pallas import tpu_sc as plsc

VOCAB = 65536
EMB = 32
ROWS = 16384
COLS = 200
N = ROWS * COLS
NW = 32
NPW = N // NW
CH = 512
PH = CH // 4
NCH = NPW // CH
GS = CH // 128

_mesh = plsc.VectorSubcoreMesh(core_axis_name="c", subcore_axis_name="s")


@functools.partial(
    pl.kernel,
    mesh=_mesh,
    compiler_params=pltpu.CompilerParams(use_tc_tiling_on_sc=False),
    out_type=jax.ShapeDtypeStruct((N // 4, 128), jnp.float32),
    scratch_types=[
        pltpu.VMEM((CH,), jnp.int32),
        pltpu.VMEM((CH,), jnp.int32),
        pltpu.VMEM((GS, 128), jnp.int32),
        pltpu.VMEM((GS, 128), jnp.int32),
        pltpu.VMEM((CH, EMB), jnp.float32),
        pltpu.VMEM((CH, EMB), jnp.float32),
        pltpu.VMEM((PH, 128), jnp.float32),
        pltpu.VMEM((PH, 128), jnp.float32),
        pltpu.SemaphoreType.DMA,
        pltpu.SemaphoreType.DMA,
        pltpu.SemaphoreType.DMA,
        pltpu.SemaphoreType.DMA,
        pltpu.SemaphoreType.DMA,
        pltpu.SemaphoreType.DMA,
    ],
)
def _gather_kernel(flags_hbm, table_hbm, emb_hbm,
                   flags_v0, flags_v1, ids_v0, ids_v1,
                   quad_v0, quad_v1, pack_v0, pack_v1,
                   sem_i0, sem_i1, sem_g0, sem_g1, sem_o0, sem_o1):
    flags_b = (flags_v0, flags_v1)
    ids_b = (ids_v0, ids_v1)
    quad_b = (quad_v0, quad_v1)
    pack_b = (pack_v0, pack_v1)
    sem_i = (sem_i0, sem_i1)
    sem_g = (sem_g0, sem_g1)
    sem_o = (sem_o0, sem_o1)

    wid = lax.axis_index("s") * 2 + lax.axis_index("c")
    wbase = wid * NPW

    def in_copy(g, p):
        return pltpu.make_async_copy(
            flags_hbm.at[pl.ds(wbase + g * CH, CH)], flags_b[p], sem_i[p])

    def gather_copy(p, j):
        return pltpu.make_async_copy(
            table_hbm.at[ids_b[p].at[j]],
            quad_b[p].at[pl.ds(j * 128, 128)], sem_g[p])

    def out_copy(g, p):
        return pltpu.make_async_copy(
            pack_b[p],
            emb_hbm.at[pl.ds((wbase + g * CH) // 4, PH)], sem_o[p])

    def fire(g, p):
        in_copy(g, p).wait()
        for j in range(GS):
            for c in range(8):
                f = flags_b[p][pl.ds(j * 128 + c * 16, 16)]
                ids_b[p][j, pl.ds(c * 16, 16)] = jnp.clip(f, 0, VOCAB - 1)
        for j in range(GS):
            gather_copy(p, j).start()

    def process(g, p):
        for j in range(GS):
            gather_copy(p, j).wait()

        def shuffle(t, carry):
            for s in range(4):
                for k in range(2):
                    pack_b[p][t, pl.ds(s * 32 + k * 16, 16)] = (
                        quad_b[p][s * PH + t, pl.ds(k * 16, 16)])
            return carry

        lax.fori_loop(0, PH, shuffle, 0)
        out_copy(g, p).start()

    in_copy(0, 0).start()

    def outer(g2, carry):
        g = g2 * 2
        fire(g, 0)
        in_copy(g + 1, 1).start()

        @pl.when(g2 >= 1)
        def _p0():
            @pl.when(g2 >= 2)
            def _w0():
                out_copy(g - 3, 1).wait()
            process(g - 1, 1)

        @pl.when(g2 <= NCH // 2 - 2)
        def _pf():
            in_copy(g + 2, 0).start()
        fire(g + 1, 1)

        @pl.when(g2 >= 1)
        def _w1():
            out_copy(g - 2, 0).wait()
        process(g, 0)
        return carry

    lax.fori_loop(0, NCH // 2, outer, 0)
    out_copy(NCH - 3, 1).wait()
    process(NCH - 1, 1)
    out_copy(NCH - 2, 0).wait()
    out_copy(NCH - 1, 1).wait()


def _emb_t_body(in_ref, out_ref):
    y = in_ref[0]
    for s in range(4):
        out_ref[0, :, s * PH:(s + 1) * PH] = y[:, s * 32:(s + 1) * 32].T


_emb_transpose = pl.pallas_call(
    _emb_t_body,
    grid=(COLS, ROWS // CH),
    in_specs=[pl.BlockSpec((1, PH, 128), lambda i, j: (i, j, 0))],
    out_specs=pl.BlockSpec((1, EMB, CH), lambda i, j: (i, 0, j)),
    out_shape=jax.ShapeDtypeStruct((COLS, EMB, ROWS), jnp.float32),
)


RPW = ROWS // NW
RB = 512
CSTRIPES = COLS // 8


@functools.partial(
    pl.kernel,
    mesh=_mesh,
    compiler_params=pltpu.CompilerParams(use_tc_tiling_on_sc=True),
    out_type=(
        jax.ShapeDtypeStruct((COLS, ROWS), jnp.float32),
        jax.ShapeDtypeStruct((COLS, 8, ROWS), jnp.float32),
    ),
    scratch_types=[
        pltpu.VMEM((8, RB), jnp.int32),
        pltpu.VMEM((8, RB), jnp.float32),
        pltpu.VMEM((8, 8, RB), jnp.float32),
    ],
)
def _maskbits_kernel(flagst_hbm, maskt_hbm, bitst_hbm, flags_v, mask_v, bits_v):
    wid = lax.axis_index("s") * 2 + lax.axis_index("c")
    rbase = wid * RPW

    def stripe(i, carry):
        c0 = i * 8
        pltpu.sync_copy(
            flagst_hbm.at[pl.ds(c0, 8), pl.ds(rbase, RB)], flags_v)
        for c in range(8):
            for v in range(RB // 16):
                f = flags_v[c, pl.ds(v * 16, 16)]
                mask_v[c, pl.ds(v * 16, 16)] = jnp.where(
                    (f & 7) == 0, 1.0, 0.0).astype(jnp.float32)
                for b in range(8):
                    bits_v[c, b, pl.ds(v * 16, 16)] = (
                        (f >> b) & 1).astype(jnp.float32)
        pltpu.sync_copy(
            mask_v, maskt_hbm.at[pl.ds(c0, 8), pl.ds(rbase, RB)])
        pltpu.sync_copy(
            bits_v, bitst_hbm.at[pl.ds(c0, 8), :, pl.ds(rbase, RB)])
        return carry

    lax.fori_loop(0, CSTRIPES, stripe, 0)


def kernel(qc_flags, emb_table):
    flags_t = qc_flags.T
    flags_c = flags_t.reshape(N)
    emb_g = _gather_kernel(flags_c, emb_table)
    emb_t = _emb_transpose(emb_g.reshape(COLS, ROWS // 4, 128))
    mask_t, bits_t = _maskbits_kernel(flags_t)
    return (mask_t.T,
            jnp.transpose(bits_t, (2, 0, 1)),
            jnp.transpose(emb_t, (2, 0, 1)))


# speedup vs baseline: 3.9813x; 3.9813x over previous
import functools

import jax
import jax.numpy as jnp
from jax import lax
from jax.experimental import pallas as pl
from jax.experimental.pallas import tpu as pltpu
from jax.experimental.pallas import tpu_sc as plsc

VOCAB = 65536
EMB = 32
ROWS = 16384
COLS = 200
N = ROWS * COLS
NW = 32
NPW = N // NW
CH = 512
PH = CH // 4
NCH = NPW // CH
GS = CH // 128

_mesh = plsc.VectorSubcoreMesh(core_axis_name="c", subcore_axis_name="s")


@functools.partial(
    pl.kernel,
    mesh=_mesh,
    compiler_params=pltpu.CompilerParams(use_tc_tiling_on_sc=False),
    out_type=jax.ShapeDtypeStruct((N // 4, 128), jnp.float32),
    scratch_types=[
        pltpu.VMEM((CH,), jnp.int32),
        pltpu.VMEM((CH,), jnp.int32),
        pltpu.VMEM((GS, 128), jnp.int32),
        pltpu.VMEM((GS, 128), jnp.int32),
        pltpu.VMEM((CH, EMB), jnp.float32),
        pltpu.VMEM((CH, EMB), jnp.float32),
        pltpu.VMEM((PH, 128), jnp.float32),
        pltpu.VMEM((PH, 128), jnp.float32),
        pltpu.SemaphoreType.DMA,
        pltpu.SemaphoreType.DMA,
        pltpu.SemaphoreType.DMA,
        pltpu.SemaphoreType.DMA,
        pltpu.SemaphoreType.DMA,
        pltpu.SemaphoreType.DMA,
    ],
)
def _gather_kernel(flags_hbm, table_hbm, emb_hbm,
                   flags_v0, flags_v1, ids_v0, ids_v1,
                   quad_v0, quad_v1, pack_v0, pack_v1,
                   sem_i0, sem_i1, sem_g0, sem_g1, sem_o0, sem_o1):
    flags_b = (flags_v0, flags_v1)
    ids_b = (ids_v0, ids_v1)
    quad_b = (quad_v0, quad_v1)
    pack_b = (pack_v0, pack_v1)
    sem_i = (sem_i0, sem_i1)
    sem_g = (sem_g0, sem_g1)
    sem_o = (sem_o0, sem_o1)

    wid = lax.axis_index("s") * 2 + lax.axis_index("c")
    wbase = wid * NPW

    def in_copy(g, p):
        return pltpu.make_async_copy(
            flags_hbm.at[pl.ds(wbase + g * CH, CH)], flags_b[p], sem_i[p])

    def gather_copy(p, j):
        return pltpu.make_async_copy(
            table_hbm.at[ids_b[p].at[j]],
            quad_b[p].at[pl.ds(j * 128, 128)], sem_g[p])

    def out_copy(g, p):
        return pltpu.make_async_copy(
            pack_b[p],
            emb_hbm.at[pl.ds((wbase + g * CH) // 4, PH)], sem_o[p])

    def fire(g, p):
        in_copy(g, p).wait()
        for j in range(GS):
            for c in range(8):
                f = flags_b[p][pl.ds(j * 128 + c * 16, 16)]
                ids_b[p][j, pl.ds(c * 16, 16)] = jnp.clip(f, 0, VOCAB - 1)
        for j in range(GS):
            gather_copy(p, j).start()

    def process(g, p):
        for j in range(GS):
            gather_copy(p, j).wait()

        def shuffle(t, carry):
            for s in range(4):
                for k in range(2):
                    pack_b[p][t, pl.ds(s * 32 + k * 16, 16)] = (
                        quad_b[p][s * PH + t, pl.ds(k * 16, 16)])
            return carry

        lax.fori_loop(0, PH, shuffle, 0)
        out_copy(g, p).start()

    in_copy(0, 0).start()

    def outer(g2, carry):
        g = g2 * 2
        fire(g, 0)
        in_copy(g + 1, 1).start()

        @pl.when(g2 >= 1)
        def _p0():
            @pl.when(g2 >= 2)
            def _w0():
                out_copy(g - 3, 1).wait()
            process(g - 1, 1)

        @pl.when(g2 <= NCH // 2 - 2)
        def _pf():
            in_copy(g + 2, 0).start()
        fire(g + 1, 1)

        @pl.when(g2 >= 1)
        def _w1():
            out_copy(g - 2, 0).wait()
        process(g, 0)
        return carry

    lax.fori_loop(0, NCH // 2, outer, 0)
    out_copy(NCH - 3, 1).wait()
    process(NCH - 1, 1)
    out_copy(NCH - 2, 0).wait()
    out_copy(NCH - 1, 1).wait()


def _emb_t_body(in_ref, out_ref):
    for b in range(ROWS // CH):
        yt = in_ref[0, b * PH:(b + 1) * PH, :].T
        for s in range(4):
            out_ref[0, :, b * CH + s * PH:b * CH + (s + 1) * PH] = (
                yt[s * 32:(s + 1) * 32, :])


_emb_transpose = pl.pallas_call(
    _emb_t_body,
    grid=(COLS,),
    in_specs=[pl.BlockSpec((1, ROWS // 4, 128), lambda i: (i, 0, 0))],
    out_specs=pl.BlockSpec((1, EMB, ROWS), lambda i: (i, 0, 0)),
    out_shape=jax.ShapeDtypeStruct((COLS, EMB, ROWS), jnp.float32),
)


RPW = ROWS // NW
RB = 512
CSTRIPES = COLS // 8


@functools.partial(
    pl.kernel,
    mesh=_mesh,
    compiler_params=pltpu.CompilerParams(use_tc_tiling_on_sc=True),
    out_type=(
        jax.ShapeDtypeStruct((COLS, ROWS), jnp.float32),
        jax.ShapeDtypeStruct((COLS, 8, ROWS), jnp.float32),
    ),
    scratch_types=[
        pltpu.VMEM((8, RB), jnp.int32),
        pltpu.VMEM((8, RB), jnp.float32),
        pltpu.VMEM((8, 8, RB), jnp.float32),
    ],
)
def _maskbits_kernel(flagst_hbm, maskt_hbm, bitst_hbm, flags_v, mask_v, bits_v):
    wid = lax.axis_index("s") * 2 + lax.axis_index("c")
    rbase = wid * RPW

    def stripe(i, carry):
        c0 = i * 8
        pltpu.sync_copy(
            flagst_hbm.at[pl.ds(c0, 8), pl.ds(rbase, RB)], flags_v)
        for c in range(8):
            for v in range(RB // 16):
                f = flags_v[c, pl.ds(v * 16, 16)]
                mask_v[c, pl.ds(v * 16, 16)] = jnp.where(
                    (f & 7) == 0, 1.0, 0.0).astype(jnp.float32)
                for b in range(8):
                    bits_v[c, b, pl.ds(v * 16, 16)] = (
                        (f >> b) & 1).astype(jnp.float32)
        pltpu.sync_copy(
            mask_v, maskt_hbm.at[pl.ds(c0, 8), pl.ds(rbase, RB)])
        pltpu.sync_copy(
            bits_v, bitst_hbm.at[pl.ds(c0, 8), :, pl.ds(rbase, RB)])
        return carry

    lax.fori_loop(0, CSTRIPES, stripe, 0)


def kernel(qc_flags, emb_table):
    flags_t = qc_flags.T
    flags_c = flags_t.reshape(N)
    emb_g = _gather_kernel(flags_c, emb_table)
    emb_t = _emb_transpose(emb_g.reshape(COLS, ROWS // 4, 128))
    mask_t, bits_t = _maskbits_kernel(flags_t)
    return (mask_t.T,
            jnp.transpose(bits_t, (2, 0, 1)),
            jnp.transpose(emb_t, (2, 0, 1)))
